# baseline (speedup 1.0000x reference)
import jax
import jax.numpy as jnp
from jax.experimental import pallas as pl
from jax.experimental.pallas import tpu as pltpu

_BLK = 2000


def _fused(x_hbm, t_ref, w1_ref, b1_ref, w2_ref, b2_ref, out_hbm,
           xb_ref, ob_ref, hf_ref, hd_ref, xsem, osem):
    n = x_hbm.shape[0]
    nb = n // _BLK

    def x_copy(b):
        return pltpu.make_async_copy(
            x_hbm.at[pl.ds(b * _BLK, _BLK), :], xb_ref.at[b % 2],
            xsem.at[b % 2])

    def o_copy(b):
        return pltpu.make_async_copy(
            ob_ref.at[b % 2], out_hbm.at[pl.ds(b * _BLK, _BLK), :],
            osem.at[b % 2])

    x_copy(0).start()
    x_copy(1).start()

    tf = (t_ref[...] != 0).astype(jnp.float32)
    eye = jnp.eye(tf.shape[0], dtype=jnp.float32)
    hf = jax.lax.dot_general(
        tf, eye, (((0,), (0,)), ((), ())),
        preferred_element_type=jnp.float32)
    hf_ref[...] = hf
    d = jnp.dot(hf, jnp.ones((tf.shape[0], 1), jnp.float32),
                preferred_element_type=jnp.float32)
    dinv = jnp.where(d > 0, 1.0 / d, 0.0)
    hd_ref[...] = dinv * hf
    bc = jax.lax.dot_general(
        jnp.ones((1, tf.shape[1]), jnp.float32), hf,
        (((1,), (0,)), ((), ())),
        preferred_element_type=jnp.float32)
    binv = jnp.where(bc > 0, 1.0 / bc, 0.0)

    s1t = jnp.zeros((x_hbm.shape[1], tf.shape[0]), jnp.float32)
    for b in range(nb):
        x_copy(b).wait()
        hb = hf_ref[pl.ds(b * _BLK, _BLK), :]
        s1t += jax.lax.dot_general(
            xb_ref[b % 2], hb, (((0,), (0,)), ((), ())),
            preferred_element_type=jnp.float32)
        if b + 2 < nb:
            x_copy(b + 2).start()

    oe1 = binv * jnp.dot(w1_ref[...], s1t,
                         preferred_element_type=jnp.float32)
    s2t = jnp.zeros((w1_ref.shape[0], tf.shape[0]), jnp.float32)
    for b in range(nb):
        hb = hf_ref[pl.ds(b * _BLK, _BLK), :]
        hdb = hd_ref[pl.ds(b * _BLK, _BLK), :]
        y = jax.lax.dot_general(
            hdb, oe1, (((1,), (1,)), ((), ())),
            preferred_element_type=jnp.float32)
        hfeat = jnp.maximum(y + b1_ref[...], 0.0)
        s2t += jax.lax.dot_general(
            hfeat, hb, (((0,), (0,)), ((), ())),
            preferred_element_type=jnp.float32)

    oe2 = binv * jnp.dot(w2_ref[...], s2t,
                         preferred_element_type=jnp.float32)
    for b in range(nb):
        if b >= 2:
            o_copy(b - 2).wait()
        hdb = hd_ref[pl.ds(b * _BLK, _BLK), :]
        y = jax.lax.dot_general(
            hdb, oe2, (((1,), (1,)), ((), ())),
            preferred_element_type=jnp.float32)
        ob_ref[b % 2] = y + b2_ref[...]
        o_copy(b).start()
    o_copy(nb - 2).wait()
    o_copy(nb - 1).wait()


def kernel(X, triplet, W1, b1, W2, b2):
    N, Din = X.shape
    E = triplet.shape[0]
    Dh = W1.shape[0]
    Dout = W2.shape[0]

    b1r = b1.reshape(1, Dh)
    b2r = b2.reshape(1, Dout)

    out = pl.pallas_call(
        _fused,
        in_specs=[
            pl.BlockSpec(memory_space=pl.ANY),
            pl.BlockSpec(memory_space=pltpu.MemorySpace.VMEM),
            pl.BlockSpec(memory_space=pltpu.MemorySpace.VMEM),
            pl.BlockSpec(memory_space=pltpu.MemorySpace.VMEM),
            pl.BlockSpec(memory_space=pltpu.MemorySpace.VMEM),
            pl.BlockSpec(memory_space=pltpu.MemorySpace.VMEM),
        ],
        out_specs=pl.BlockSpec(memory_space=pl.ANY),
        out_shape=jax.ShapeDtypeStruct((N, Dout), jnp.float32),
        scratch_shapes=[
            pltpu.VMEM((2, _BLK, Din), jnp.float32),
            pltpu.VMEM((2, _BLK, Dout), jnp.float32),
            pltpu.VMEM((N, E), jnp.float32),
            pltpu.VMEM((N, E), jnp.float32),
            pltpu.SemaphoreType.DMA((2,)),
            pltpu.SemaphoreType.DMA((2,)),
        ],
    )(X, triplet, W1, b1r, W2, b2r)

    return out


# speedup vs baseline: 1.0288x; 1.0288x over previous
import jax
import jax.numpy as jnp
from jax.experimental import pallas as pl
from jax.experimental.pallas import tpu as pltpu

_BLK = 2000


def _fused(x_hbm, t_ref, w1_ref, b1_ref, w2_ref, b2_ref, out_hbm,
           xb_ref, ob_ref, hf_ref, dinv_ref, xsem, osem):
    n = x_hbm.shape[0]
    nb = n // _BLK

    def x_copy(b):
        return pltpu.make_async_copy(
            x_hbm.at[pl.ds(b * _BLK, _BLK), :], xb_ref.at[b % 2],
            xsem.at[b % 2])

    def o_copy(b):
        return pltpu.make_async_copy(
            ob_ref.at[b % 2], out_hbm.at[pl.ds(b * _BLK, _BLK), :],
            osem.at[b % 2])

    x_copy(0).start()
    x_copy(1).start()

    tf = (t_ref[...] != 0).astype(jnp.float32)
    eye = jnp.eye(tf.shape[0], dtype=jnp.float32)
    hf = jax.lax.dot_general(
        tf, eye, (((0,), (0,)), ((), ())),
        preferred_element_type=jnp.float32)
    hf_ref[...] = hf
    d = jnp.dot(hf, jnp.ones((tf.shape[0], 1), jnp.float32),
                preferred_element_type=jnp.float32)
    dinv_ref[...] = jnp.where(d > 0, 1.0 / d, 0.0)
    bc = jax.lax.dot_general(
        jnp.ones((1, tf.shape[1]), jnp.float32), hf,
        (((1,), (0,)), ((), ())),
        preferred_element_type=jnp.float32)
    binv = jnp.where(bc > 0, 1.0 / bc, 0.0)

    s1t = jnp.zeros((x_hbm.shape[1], tf.shape[0]), jnp.float32)
    for b in range(nb):
        x_copy(b).wait()
        hb = hf_ref[pl.ds(b * _BLK, _BLK), :]
        s1t += jax.lax.dot_general(
            xb_ref[b % 2], hb, (((0,), (0,)), ((), ())),
            preferred_element_type=jnp.float32)
        if b + 2 < nb:
            x_copy(b + 2).start()

    oe1 = binv * jnp.dot(w1_ref[...], s1t,
                         preferred_element_type=jnp.float32)
    s2t = jnp.zeros((w1_ref.shape[0], tf.shape[0]), jnp.float32)
    for b in range(nb):
        hb = hf_ref[pl.ds(b * _BLK, _BLK), :]
        y = jax.lax.dot_general(
            hb, oe1, (((1,), (1,)), ((), ())),
            preferred_element_type=jnp.float32)
        dinv = dinv_ref[pl.ds(b * _BLK, _BLK), :]
        hfeat = jnp.maximum(dinv * y + b1_ref[...], 0.0)
        s2t += jax.lax.dot_general(
            hfeat, hb, (((0,), (0,)), ((), ())),
            preferred_element_type=jnp.float32)

    oe2 = binv * jnp.dot(w2_ref[...], s2t,
                         preferred_element_type=jnp.float32)
    for b in range(nb):
        if b >= 2:
            o_copy(b - 2).wait()
        hb = hf_ref[pl.ds(b * _BLK, _BLK), :]
        y = jax.lax.dot_general(
            hb, oe2, (((1,), (1,)), ((), ())),
            preferred_element_type=jnp.float32)
        dinv = dinv_ref[pl.ds(b * _BLK, _BLK), :]
        ob_ref[b % 2] = dinv * y + b2_ref[...]
        o_copy(b).start()
    o_copy(nb - 2).wait()
    o_copy(nb - 1).wait()


def kernel(X, triplet, W1, b1, W2, b2):
    N, Din = X.shape
    E = triplet.shape[0]
    Dh = W1.shape[0]
    Dout = W2.shape[0]

    b1r = b1.reshape(1, Dh)
    b2r = b2.reshape(1, Dout)

    out = pl.pallas_call(
        _fused,
        in_specs=[
            pl.BlockSpec(memory_space=pl.ANY),
            pl.BlockSpec(memory_space=pltpu.MemorySpace.VMEM),
            pl.BlockSpec(memory_space=pltpu.MemorySpace.VMEM),
            pl.BlockSpec(memory_space=pltpu.MemorySpace.VMEM),
            pl.BlockSpec(memory_space=pltpu.MemorySpace.VMEM),
            pl.BlockSpec(memory_space=pltpu.MemorySpace.VMEM),
        ],
        out_specs=pl.BlockSpec(memory_space=pl.ANY),
        out_shape=jax.ShapeDtypeStruct((N, Dout), jnp.float32),
        scratch_shapes=[
            pltpu.VMEM((2, _BLK, Din), jnp.float32),
            pltpu.VMEM((2, _BLK, Dout), jnp.float32),
            pltpu.VMEM((N, E), jnp.float32),
            pltpu.VMEM((N, 1), jnp.float32),
            pltpu.SemaphoreType.DMA((2,)),
            pltpu.SemaphoreType.DMA((2,)),
        ],
    )(X, triplet, W1, b1r, W2, b2r)

    return out


# speedup vs baseline: 1.0557x; 1.0262x over previous
import jax
import jax.numpy as jnp
from jax.experimental import pallas as pl
from jax.experimental.pallas import tpu as pltpu

_BLK = 5000


def _fused(x_hbm, t_ref, w1_ref, b1_ref, w2_ref, b2_ref, out_hbm,
           xb_ref, ob_ref, hf_ref, dinv_ref, xsem, osem):
    n = x_hbm.shape[0]
    nb = n // _BLK

    def x_copy(b):
        return pltpu.make_async_copy(
            x_hbm.at[pl.ds(b * _BLK, _BLK), :], xb_ref.at[b % 2],
            xsem.at[b % 2])

    def o_copy(b):
        return pltpu.make_async_copy(
            ob_ref.at[b % 2], out_hbm.at[pl.ds(b * _BLK, _BLK), :],
            osem.at[b % 2])

    x_copy(0).start()
    x_copy(1).start()

    tf = (t_ref[...] != 0).astype(jnp.float32)
    eye = jnp.eye(tf.shape[0], dtype=jnp.float32)
    hf = jax.lax.dot_general(
        tf, eye, (((0,), (0,)), ((), ())),
        preferred_element_type=jnp.float32)
    hf_ref[...] = hf
    d = jnp.dot(hf, jnp.ones((tf.shape[0], 1), jnp.float32),
                preferred_element_type=jnp.float32)
    dinv_ref[...] = jnp.where(d > 0, 1.0 / d, 0.0)
    bc = jax.lax.dot_general(
        jnp.ones((1, tf.shape[1]), jnp.float32), hf,
        (((1,), (0,)), ((), ())),
        preferred_element_type=jnp.float32)
    binv = jnp.where(bc > 0, 1.0 / bc, 0.0)

    s1t = jnp.zeros((x_hbm.shape[1], tf.shape[0]), jnp.float32)
    for b in range(nb):
        x_copy(b).wait()
        hb = hf_ref[pl.ds(b * _BLK, _BLK), :]
        s1t += jax.lax.dot_general(
            xb_ref[b % 2], hb, (((0,), (0,)), ((), ())),
            preferred_element_type=jnp.float32)
        if b + 2 < nb:
            x_copy(b + 2).start()

    oe1 = binv * jnp.dot(w1_ref[...], s1t,
                         preferred_element_type=jnp.float32)
    s2t = jnp.zeros((w1_ref.shape[0], tf.shape[0]), jnp.float32)
    for b in range(nb):
        hb = hf_ref[pl.ds(b * _BLK, _BLK), :]
        y = jax.lax.dot_general(
            hb, oe1, (((1,), (1,)), ((), ())),
            preferred_element_type=jnp.float32)
        dinv = dinv_ref[pl.ds(b * _BLK, _BLK), :]
        hfeat = jnp.maximum(dinv * y + b1_ref[...], 0.0)
        s2t += jax.lax.dot_general(
            hfeat, hb, (((0,), (0,)), ((), ())),
            preferred_element_type=jnp.float32)

    oe2 = binv * jnp.dot(w2_ref[...], s2t,
                         preferred_element_type=jnp.float32)
    for b in range(nb):
        if b >= 2:
            o_copy(b - 2).wait()
        hb = hf_ref[pl.ds(b * _BLK, _BLK), :]
        y = jax.lax.dot_general(
            hb, oe2, (((1,), (1,)), ((), ())),
            preferred_element_type=jnp.float32)
        dinv = dinv_ref[pl.ds(b * _BLK, _BLK), :]
        ob_ref[b % 2] = dinv * y + b2_ref[...]
        o_copy(b).start()
    o_copy(nb - 2).wait()
    o_copy(nb - 1).wait()


def kernel(X, triplet, W1, b1, W2, b2):
    N, Din = X.shape
    E = triplet.shape[0]
    Dh = W1.shape[0]
    Dout = W2.shape[0]

    b1r = b1.reshape(1, Dh)
    b2r = b2.reshape(1, Dout)

    out = pl.pallas_call(
        _fused,
        in_specs=[
            pl.BlockSpec(memory_space=pl.ANY),
            pl.BlockSpec(memory_space=pltpu.MemorySpace.VMEM),
            pl.BlockSpec(memory_space=pltpu.MemorySpace.VMEM),
            pl.BlockSpec(memory_space=pltpu.MemorySpace.VMEM),
            pl.BlockSpec(memory_space=pltpu.MemorySpace.VMEM),
            pl.BlockSpec(memory_space=pltpu.MemorySpace.VMEM),
        ],
        out_specs=pl.BlockSpec(memory_space=pl.ANY),
        out_shape=jax.ShapeDtypeStruct((N, Dout), jnp.float32),
        scratch_shapes=[
            pltpu.VMEM((2, _BLK, Din), jnp.float32),
            pltpu.VMEM((2, _BLK, Dout), jnp.float32),
            pltpu.VMEM((N, E), jnp.float32),
            pltpu.VMEM((N, 1), jnp.float32),
            pltpu.SemaphoreType.DMA((2,)),
            pltpu.SemaphoreType.DMA((2,)),
        ],
    )(X, triplet, W1, b1r, W2, b2r)

    return out


# speedup vs baseline: 1.1255x; 1.0661x over previous
import jax
import jax.numpy as jnp
from jax.experimental import pallas as pl
from jax.experimental.pallas import tpu as pltpu

_BLK = 5000


def _fused(x_ref, t_ref, w1_ref, b1_ref, w2_ref, b2_ref, out_ref,
           hf_ref, dinv_ref, s1t_ref, bc_ref, s2t_ref, oe_ref):
    s = pl.program_id(0)
    i = pl.program_id(1)

    @pl.when(jnp.logical_and(s == 0, i == 0))
    def _():
        tf = (t_ref[...] != 0).astype(jnp.float32)
        eye = jnp.eye(tf.shape[0], dtype=jnp.float32)
        hf_ref[...] = jax.lax.dot_general(
            tf, eye, (((0,), (0,)), ((), ())),
            preferred_element_type=jnp.float32)
        bc_ref[...] = jnp.zeros_like(bc_ref)
        s1t_ref[...] = jnp.zeros_like(s1t_ref)

    hb = hf_ref[pl.ds(i * _BLK, _BLK), :]

    @pl.when(s == 0)
    def _():
        s1t_ref[...] += jax.lax.dot_general(
            x_ref[...], hb, (((0,), (0,)), ((), ())),
            preferred_element_type=jnp.float32)
        bc_ref[...] += jnp.sum(hb, axis=0, keepdims=True)

    @pl.when(jnp.logical_and(s == 1, i == 0))
    def _():
        bc = bc_ref[...]
        binv = jnp.where(bc > 0, 1.0 / bc, 0.0)
        oe_ref[...] = binv * jnp.dot(
            w1_ref[...], s1t_ref[...],
            preferred_element_type=jnp.float32)
        s2t_ref[...] = jnp.zeros_like(s2t_ref)

    @pl.when(s == 1)
    def _():
        d = jnp.sum(hb, axis=1, keepdims=True)
        dinv = jnp.where(d > 0, 1.0 / d, 0.0)
        dinv_ref[pl.ds(i * _BLK, _BLK), :] = dinv
        y = jax.lax.dot_general(
            hb, oe_ref[...], (((1,), (1,)), ((), ())),
            preferred_element_type=jnp.float32)
        hfeat = jnp.maximum(dinv * y + b1_ref[...], 0.0)
        s2t_ref[...] += jax.lax.dot_general(
            hfeat, hb, (((0,), (0,)), ((), ())),
            preferred_element_type=jnp.float32)

    @pl.when(jnp.logical_and(s == 2, i == 0))
    def _():
        bc = bc_ref[...]
        binv = jnp.where(bc > 0, 1.0 / bc, 0.0)
        oe_ref[...] = binv * jnp.dot(
            w2_ref[...], s2t_ref[...],
            preferred_element_type=jnp.float32)

    @pl.when(s == 2)
    def _():
        y = jax.lax.dot_general(
            hb, oe_ref[...], (((1,), (1,)), ((), ())),
            preferred_element_type=jnp.float32)
        out_ref[...] = dinv_ref[pl.ds(i * _BLK, _BLK), :] * y + b2_ref[...]


def kernel(X, triplet, W1, b1, W2, b2):
    N, Din = X.shape
    E = triplet.shape[0]
    Dh = W1.shape[0]
    Dout = W2.shape[0]
    nb = N // _BLK

    b1r = b1.reshape(1, Dh)
    b2r = b2.reshape(1, Dout)

    out = pl.pallas_call(
        _fused,
        grid=(3, nb),
        in_specs=[
            pl.BlockSpec((_BLK, Din),
                         lambda s, i: (jnp.where(s == 0, i, nb - 1), 0)),
            pl.BlockSpec((E, N), lambda s, i: (0, 0)),
            pl.BlockSpec((Dh, Din), lambda s, i: (0, 0)),
            pl.BlockSpec((1, Dh), lambda s, i: (0, 0)),
            pl.BlockSpec((Dout, Dh), lambda s, i: (0, 0)),
            pl.BlockSpec((1, Dout), lambda s, i: (0, 0)),
        ],
        out_specs=pl.BlockSpec((_BLK, Dout),
                               lambda s, i: (jnp.where(s == 2, i, 0), 0)),
        out_shape=jax.ShapeDtypeStruct((N, Dout), jnp.float32),
        scratch_shapes=[
            pltpu.VMEM((N, E), jnp.float32),
            pltpu.VMEM((N, 1), jnp.float32),
            pltpu.VMEM((Din, E), jnp.float32),
            pltpu.VMEM((1, E), jnp.float32),
            pltpu.VMEM((Dh, E), jnp.float32),
            pltpu.VMEM((max(Dh, Dout), E), jnp.float32),
        ],
    )(X, triplet, W1, b1r, W2, b2r)

    return out
